# Initial kernel scaffold; baseline (speedup 1.0000x reference)
import jax
import jax.numpy as jnp
from jax.experimental import pallas as pl


def kernel(node_feat, node_opcode, edge_index, node_config_feat, node_config_ids, config_edge_index, params):
    raise NotImplementedError("write your pallas kernel here")


# speedup vs baseline: 5.8860x; 5.8860x over previous
import functools

import jax
import jax.numpy as jnp
from jax import lax
from jax.experimental import pallas as pl
from jax.experimental.pallas import tpu as pltpu
from jax.experimental.pallas import tpu_sc as plsc

_NC = 2
_NS = 16
_NW = _NC * _NS


@functools.lru_cache(maxsize=None)
def _mesh():
    return plsc.VectorSubcoreMesh(core_axis_name="c", subcore_axis_name="s",
                                  num_cores=_NC, num_subcores=_NS)


def _leaky(x):
    return jnp.where(x >= 0, x, 0.01 * x)


def _dot(a, b):
    ah = a.astype(jnp.bfloat16)
    al = (a - ah.astype(jnp.float32)).astype(jnp.bfloat16)
    bh = b.astype(jnp.bfloat16)
    bl = (b - bh.astype(jnp.float32)).astype(jnp.bfloat16)

    def d(x, y):
        return jnp.dot(x, y, preferred_element_type=jnp.float32)

    return d(ah, bh) + d(ah, bl) + d(al, bh)


def _dot_exact_lhs(a_bf16, b):
    bh = b.astype(jnp.bfloat16)
    bl = (b - bh.astype(jnp.float32)).astype(jnp.bfloat16)

    def d(x, y):
        return jnp.dot(x, y, preferred_element_type=jnp.float32)

    return d(a_bf16, bh) + d(a_bf16, bl)


def _seg_sum_call(y, src, dst, with_cnt):
    R, F = y.shape
    E = src.shape[0]
    K = 400
    EW = E // _NW
    CH = EW // K
    RT = R // _NS

    zr = jnp.zeros((R, F), jnp.float32)
    outs = [jax.ShapeDtypeStruct((_NC * R, F), jnp.float32)]
    scratch = [
        pltpu.VMEM((K,), jnp.int32),
        pltpu.VMEM((K,), jnp.int32),
        pltpu.VMEM((K, F), jnp.float32),
        pltpu.VMEM_SHARED((R, F), jnp.float32),
        pltpu.SemaphoreType.DMA,
    ]

    if with_cnt:
        zc = jnp.zeros((R, 16), jnp.float32)
        ones = jnp.ones((K, 16), jnp.float32)
        outs.append(jax.ShapeDtypeStruct((_NC * R, 16), jnp.float32))
        scratch += [
            pltpu.VMEM((K, 16), jnp.float32),
            pltpu.VMEM_SHARED((R, 16), jnp.float32),
        ]

        def body(y_h, s_h, d_h, zr_h, zc_h, on_h, out_h, cnt_h,
                 idx_s, idx_d, rows, acc, sem,
                 ones_v, cacc):
            cid = lax.axis_index("c")
            sid = lax.axis_index("s")
            wid = cid * _NS + sid
            sl = pl.ds(sid * RT, RT)
            pltpu.sync_copy(zr_h.at[sl], acc.at[sl])
            pltpu.sync_copy(zc_h.at[sl], cacc.at[sl])
            pltpu.sync_copy(on_h, ones_v)
            plsc.subcore_barrier()
            base = wid * EW

            def step(i, t):
                off = base + i * K
                pltpu.sync_copy(s_h.at[pl.ds(off, K)], idx_s)
                pltpu.sync_copy(d_h.at[pl.ds(off, K)], idx_d)
                pltpu.async_copy(y_h.at[idx_s], rows, sem).wait()
                pltpu.sync_copy(rows, acc.at[idx_d], add=True)
                pltpu.sync_copy(ones_v, cacc.at[idx_d], add=True)
                return t

            lax.fori_loop(0, CH, step, jnp.int32(0))
            plsc.subcore_barrier()
            pltpu.sync_copy(acc.at[sl], out_h.at[pl.ds(cid * R + sid * RT, RT)])
            pltpu.sync_copy(cacc.at[sl], cnt_h.at[pl.ds(cid * R + sid * RT, RT)])

        fn = pl.kernel(body, out_type=outs, mesh=_mesh(), scratch_types=scratch,
                       compiler_params=pltpu.CompilerParams(use_tc_tiling_on_sc=False))
        out, cnt = fn(y, src, dst, zr, zc, ones)
        return out.reshape(_NC, R, F), cnt.reshape(_NC, R, 16)[:, :, :1]

    def body(y_h, s_h, d_h, zr_h, out_h,
             idx_s, idx_d, rows, acc, sem):
        cid = lax.axis_index("c")
        sid = lax.axis_index("s")
        wid = cid * _NS + sid
        sl = pl.ds(sid * RT, RT)
        pltpu.sync_copy(zr_h.at[sl], acc.at[sl])
        plsc.subcore_barrier()
        base = wid * EW

        def step(i, t):
            off = base + i * K
            pltpu.sync_copy(s_h.at[pl.ds(off, K)], idx_s)
            pltpu.sync_copy(d_h.at[pl.ds(off, K)], idx_d)
            pltpu.async_copy(y_h.at[idx_s], rows, sem).wait()
            pltpu.sync_copy(rows, acc.at[idx_d], add=True)
            return t

        lax.fori_loop(0, CH, step, jnp.int32(0))
        plsc.subcore_barrier()
        pltpu.sync_copy(acc.at[sl], out_h.at[pl.ds(cid * R + sid * RT, RT)])

    fn = pl.kernel(body, out_type=outs, mesh=_mesh(), scratch_types=scratch,
                   compiler_params=pltpu.CompilerParams(use_tc_tiling_on_sc=False))
    (out,) = fn(y, src, dst, zr)
    return out.reshape(_NC, R, F)


def _gather2_call(t1, t2, idx):
    R, F = t1.shape
    B = idx.shape[0]
    PW = B // _NW
    RT = R // _NS

    def body(t1_h, t2_h, id_h, o1_h, o2_h, idx_v, rows, tab, sem):
        cid = lax.axis_index("c")
        sid = lax.axis_index("s")
        wid = cid * _NS + sid
        sl = pl.ds(sid * RT, RT)
        pltpu.sync_copy(id_h.at[pl.ds(wid * PW, PW)], idx_v)
        pltpu.sync_copy(t1_h.at[sl], tab.at[sl])
        plsc.subcore_barrier()
        pltpu.async_copy(tab.at[idx_v], rows, sem).wait()
        pltpu.sync_copy(rows, o1_h.at[pl.ds(wid * PW, PW)])
        plsc.subcore_barrier()
        pltpu.sync_copy(t2_h.at[sl], tab.at[sl])
        plsc.subcore_barrier()
        pltpu.async_copy(tab.at[idx_v], rows, sem).wait()
        pltpu.sync_copy(rows, o2_h.at[pl.ds(wid * PW, PW)])

    outs = [jax.ShapeDtypeStruct((B, F), jnp.float32),
            jax.ShapeDtypeStruct((B, F), jnp.float32)]
    scratch = [
        pltpu.VMEM((PW,), jnp.int32),
        pltpu.VMEM((PW, F), jnp.float32),
        pltpu.VMEM_SHARED((R, F), jnp.float32),
        pltpu.SemaphoreType.DMA,
    ]
    fn = pl.kernel(body, out_type=outs, mesh=_mesh(), scratch_types=scratch,
                   compiler_params=pltpu.CompilerParams(use_tc_tiling_on_sc=False))
    return fn(t1, t2, idx)


_NB = 10
_BR = 1024
_FA = 96
_FB = 80


def _tc_pre_call(node_feat, opcode_r, emb_pad, wr0):
    def body(nf_ref, op_ref, emb_ref, wr_ref, xa_ref, xb_ref, v_ref):
        op = op_ref[0, 0, :]
        oh = (op[:, None] == lax.broadcasted_iota(jnp.int32, (_BR, 128), 1))
        er = _dot(oh.astype(jnp.float32), emb_ref[...])
        nf = nf_ref[...]
        x0 = jnp.concatenate([nf, er], axis=1)
        xa_ref[...] = nf[:, :_FA]
        xb_ref[...] = jnp.concatenate(
            [nf[:, _FA:], er, jnp.zeros((_BR, 4), jnp.float32)], axis=1)
        v_ref[...] = _dot(x0, wr_ref[...])

    n = node_feat.shape[0]
    fdim = node_feat.shape[1]
    return pl.pallas_call(
        body,
        grid=(_NB,),
        in_specs=[
            pl.BlockSpec((_BR, fdim), lambda i: (i, 0)),
            pl.BlockSpec((1, 1, _BR), lambda i: (i, 0, 0)),
            pl.BlockSpec((128, 32), lambda i: (0, 0)),
            pl.BlockSpec((fdim + 32, 64), lambda i: (0, 0)),
        ],
        out_specs=[pl.BlockSpec((_BR, _FA), lambda i: (i, 0)),
                   pl.BlockSpec((_BR, _FB), lambda i: (i, 0)),
                   pl.BlockSpec((_BR, 64), lambda i: (i, 0))],
        out_shape=[jax.ShapeDtypeStruct((n, _FA), jnp.float32),
                   jax.ShapeDtypeStruct((n, _FB), jnp.float32),
                   jax.ShapeDtypeStruct((n, 64), jnp.float32)],
    )(node_feat, opcode_r, emb_pad, wr0)


def _tc_layer1_call(sa, sb, cnt_part, v0, wl0, bl0, wr1):
    def body(sa_ref, sb_ref, c_ref, v_ref, wl_ref, b_ref, wr_ref,
             x_ref, vn_ref):
        m = jnp.concatenate([sa_ref[0] + sa_ref[1], sb_ref[0] + sb_ref[1]],
                            axis=1)
        c = jnp.maximum(c_ref[0] + c_ref[1], 1.0)
        m = m / c
        x = _leaky(_dot(m, wl_ref[...]) + b_ref[...] + v_ref[...])
        x_ref[...] = x
        vn_ref[...] = _dot(x, wr_ref[...])

    n = v0.shape[0]
    return pl.pallas_call(
        body,
        grid=(_NB,),
        in_specs=[
            pl.BlockSpec((_NC, _BR, _FA), lambda i: (0, i, 0)),
            pl.BlockSpec((_NC, _BR, _FB), lambda i: (0, i, 0)),
            pl.BlockSpec((_NC, _BR, 1), lambda i: (0, i, 0)),
            pl.BlockSpec((_BR, 64), lambda i: (i, 0)),
            pl.BlockSpec((_FA + _FB, 64), lambda i: (0, 0)),
            pl.BlockSpec((1, 64), lambda i: (0, 0)),
            pl.BlockSpec((64, 64), lambda i: (0, 0)),
        ],
        out_specs=[pl.BlockSpec((_BR, 64), lambda i: (i, 0)),
                   pl.BlockSpec((_BR, 64), lambda i: (i, 0))],
        out_shape=[jax.ShapeDtypeStruct((n, 64), jnp.float32),
                   jax.ShapeDtypeStruct((n, 64), jnp.float32)],
    )(sa, sb, cnt_part, v0, wl0, bl0, wr1)


def _tc_layer_call(s_part, cnt_part, v_prev, wl, bl, wr_next, last):
    def body_mid(s_ref, c_ref, vp_ref, wl_ref, b_ref, wr_ref, x_ref, v_ref):
        m = (s_ref[0] + s_ref[1]) / jnp.maximum(c_ref[0] + c_ref[1], 1.0)
        x = _leaky(_dot(m, wl_ref[...]) + b_ref[...] + vp_ref[...])
        x_ref[...] = x
        v_ref[...] = _dot(x, wr_ref[...])

    def body_last(s_ref, c_ref, vp_ref, wl_ref, b_ref, wr_ref, x_ref):
        m = (s_ref[0] + s_ref[1]) / jnp.maximum(c_ref[0] + c_ref[1], 1.0)
        x_ref[...] = _leaky(_dot(m, wl_ref[...]) + b_ref[...] + vp_ref[...])

    n = v_prev.shape[0]
    in_specs = [
        pl.BlockSpec((_NC, _BR, 64), lambda i: (0, i, 0)),
        pl.BlockSpec((_NC, _BR, 1), lambda i: (0, i, 0)),
        pl.BlockSpec((_BR, 64), lambda i: (i, 0)),
        pl.BlockSpec((64, 64), lambda i: (0, 0)),
        pl.BlockSpec((1, 64), lambda i: (0, 0)),
        pl.BlockSpec((64, 64), lambda i: (0, 0)),
    ]
    if last:
        return pl.pallas_call(
            body_last,
            grid=(_NB,),
            in_specs=in_specs,
            out_specs=pl.BlockSpec((_BR, 64), lambda i: (i, 0)),
            out_shape=jax.ShapeDtypeStruct((n, 64), jnp.float32),
        )(s_part, cnt_part, v_prev, wl, bl, wr_next)
    return pl.pallas_call(
        body_mid,
        grid=(_NB,),
        in_specs=in_specs,
        out_specs=[pl.BlockSpec((_BR, 64), lambda i: (i, 0)),
                   pl.BlockSpec((_BR, 64), lambda i: (i, 0))],
        out_shape=[jax.ShapeDtypeStruct((n, 64), jnp.float32),
                   jax.ShapeDtypeStruct((n, 64), jnp.float32)],
    )(s_part, cnt_part, v_prev, wl, bl, wr_next)


def _tc_mean_call(s_part, cnt_part):
    def body(s_ref, c_ref, o_ref):
        s = s_ref[0] + s_ref[1]
        c = jnp.maximum(c_ref[0] + c_ref[1], 1.0)
        o_ref[...] = s / c

    n = s_part.shape[1]
    return pl.pallas_call(
        body,
        grid=(_NB,),
        in_specs=[pl.BlockSpec((_NC, _BR, 64), lambda i: (0, i, 0)),
                  pl.BlockSpec((_NC, _BR, 1), lambda i: (0, i, 0))],
        out_specs=pl.BlockSpec((_BR, 64), lambda i: (i, 0)),
        out_shape=jax.ShapeDtypeStruct((n, 64), jnp.float32),
    )(s_part, cnt_part)


def _tc_acfg_call(ce_r, nc):
    ne = ce_r.shape[2]

    def body(ce_ref, a_ref, c_ref):
        s = ce_ref[0, 0, :]
        d = ce_ref[1, 0, :]
        ohd = (d[None, :] == lax.broadcasted_iota(jnp.int32, (nc, ne), 0))
        ohs = (s[:, None] == lax.broadcasted_iota(jnp.int32, (ne, nc), 1))
        a = jnp.dot(ohd.astype(jnp.bfloat16), ohs.astype(jnp.bfloat16),
                    preferred_element_type=jnp.float32)
        c_ref[...] = jnp.maximum(jnp.sum(a, axis=1, keepdims=True), 1.0)
        a_ref[...] = a.astype(jnp.bfloat16)

    return pl.pallas_call(
        body,
        in_specs=[pl.BlockSpec((2, 1, ne), lambda: (0, 0, 0))],
        out_specs=[pl.BlockSpec((nc, nc), lambda: (0, 0)),
                   pl.BlockSpec((nc, 1), lambda: (0, 0))],
        out_shape=[jax.ShapeDtypeStruct((nc, nc), jnp.bfloat16),
                   jax.ShapeDtypeStruct((nc, 1), jnp.float32)],
    )(ce_r)


def _tc_cfg_nbr_call(cn0, a_n, cnt, layers):
    nc = cn0.shape[0]

    def body(cn_ref, a_ref, c_ref, w1l, w1r, b1, w2l, w2r, b2, w3l, w3r, b3,
             o_ref):
        a = a_ref[...]
        c = c_ref[...]
        cn = cn_ref[...]
        for wl_ref, wr_ref, b_ref in ((w1l, w1r, b1), (w2l, w2r, b2),
                                      (w3l, w3r, b3)):
            m = _dot_exact_lhs(a, cn) / c
            cn = _leaky(_dot(m, wl_ref[...]) + b_ref[...] +
                        _dot(cn, wr_ref[...]))
        o_ref[...] = cn

    w_in = []
    w_specs = []
    for wl, wr, b in layers:
        w_in += [wl, wr, b]
        w_specs += [pl.BlockSpec((64, 64), lambda: (0, 0)),
                    pl.BlockSpec((64, 64), lambda: (0, 0)),
                    pl.BlockSpec((1, 64), lambda: (0, 0))]
    return pl.pallas_call(
        body,
        in_specs=[pl.BlockSpec((nc, 64), lambda: (0, 0)),
                  pl.BlockSpec((nc, nc), lambda: (0, 0)),
                  pl.BlockSpec((nc, 1), lambda: (0, 0))] + w_specs,
        out_specs=pl.BlockSpec((nc, 64), lambda: (0, 0)),
        out_shape=jax.ShapeDtypeStruct((nc, 64), jnp.float32),
    )(cn0, a_n, cnt, *w_in)


def _tc_config_head_call(ncf, cn, xs, prjw, prjb, a_n, cnt, layers,
                         d1, d2, d3):
    c, nc, fin = ncf.shape

    def body(ncf_ref, cn_ref, xs_ref, pw_ref, pb_ref, a_ref, c_ref,
             w1l, w1r, b1, w2l, w2r, b2, w3l, w3r, b3,
             d1_ref, d2_ref, d3_ref, o_ref):
        p = _leaky(_dot(ncf_ref[0], pw_ref[...]) + pb_ref[...])
        h = jnp.concatenate([cn_ref[...], xs_ref[...], p], axis=1)
        h = h / jnp.maximum(jnp.sqrt(jnp.sum(h * h, axis=1, keepdims=True)),
                            1e-12)
        a = a_ref[...]
        cdeg = c_ref[...]
        for wl_ref, wr_ref, b_ref in ((w1l, w1r, b1), (w2l, w2r, b2),
                                      (w3l, w3r, b3)):
            m = _dot_exact_lhs(a, h) / cdeg
            h = _leaky(_dot(m, wl_ref[...]) + b_ref[...] +
                       _dot(h, wr_ref[...]))
        pooled = jnp.mean(h, axis=0, keepdims=True)
        y = _leaky(_dot(pooled, d1_ref[...]))
        y = _leaky(_dot(y, d2_ref[...]))
        y = _dot(y, d3_ref[...])
        o_ref[...] = jnp.broadcast_to(y[:, :, None], (1, 1, 128))

    w_in = []
    w_specs = []
    for wl, wr, b in layers:
        kdim = wl.shape[0]
        w_in += [wl, wr, b]
        w_specs += [pl.BlockSpec((kdim, 64), lambda i: (0, 0)),
                    pl.BlockSpec((kdim, 64), lambda i: (0, 0)),
                    pl.BlockSpec((1, 64), lambda i: (0, 0))]
    return pl.pallas_call(
        body,
        grid=(c,),
        in_specs=[
            pl.BlockSpec((1, nc, fin), lambda i: (i, 0, 0)),
            pl.BlockSpec((nc, 64), lambda i: (0, 0)),
            pl.BlockSpec((nc, 64), lambda i: (0, 0)),
            pl.BlockSpec((fin, 64), lambda i: (0, 0)),
            pl.BlockSpec((1, 64), lambda i: (0, 0)),
            pl.BlockSpec((nc, nc), lambda i: (0, 0)),
            pl.BlockSpec((nc, 1), lambda i: (0, 0)),
        ] + w_specs + [
            pl.BlockSpec((64, 64), lambda i: (0, 0)),
            pl.BlockSpec((64, 64), lambda i: (0, 0)),
            pl.BlockSpec((64, 1), lambda i: (0, 0)),
        ],
        out_specs=pl.BlockSpec((1, 1, 128), lambda i: (i, 0, 0)),
        out_shape=jax.ShapeDtypeStruct((c, 1, 128), jnp.float32),
    )(ncf, cn, xs, prjw, prjb, a_n, cnt, *w_in, d1, d2, d3)


def kernel(node_feat, node_opcode, edge_index, node_config_feat,
           node_config_ids, config_edge_index, params):
    n = node_feat.shape[0]
    c, nc, fin = node_config_feat.shape
    npad = _NB * _BR

    src = edge_index[0]
    dst = edge_index[1]
    node_feat = jnp.pad(node_feat, ((0, npad - n), (0, 0)))
    opcode_r = jnp.pad(node_opcode.astype(jnp.int32),
                       (0, npad - n)).reshape(_NB, 1, _BR)
    emb_pad = jnp.pad(params["emb"], ((0, 8), (0, 0)))
    ids_pad = jnp.concatenate(
        [node_config_ids.astype(jnp.int32),
         jnp.zeros((24,), jnp.int32)])
    ce_r = config_edge_index.astype(jnp.int32).reshape(2, 1, -1)

    g = params["node_gnn"]
    wl0 = jnp.pad(g[0]["Wl"].T, ((0, _FA + _FB - g[0]["Wl"].shape[1]), (0, 0)))
    xa, xb, v0 = _tc_pre_call(node_feat, opcode_r, emb_pad, g[0]["Wr"].T)

    sb, cnt = _seg_sum_call(xb, src, dst, with_cnt=True)
    sa = _seg_sum_call(xa, src, dst, with_cnt=False)
    x1, v1 = _tc_layer1_call(sa, sb, cnt, v0, wl0,
                             g[0]["bl"].reshape(1, -1), g[1]["Wr"].T)
    s1 = _seg_sum_call(x1, src, dst, with_cnt=False)
    x2, v2 = _tc_layer_call(s1, cnt, v1, g[1]["Wl"].T,
                            g[1]["bl"].reshape(1, -1), g[2]["Wr"].T, last=False)
    s2 = _seg_sum_call(x2, src, dst, with_cnt=False)
    x3 = _tc_layer_call(s2, cnt, v2, g[2]["Wl"].T,
                        g[2]["bl"].reshape(1, -1), g[2]["Wr"].T, last=True)

    s3 = _seg_sum_call(x3, src, dst, with_cnt=False)
    agg4 = _tc_mean_call(s3, cnt)
    xs_pad, cn0_pad = _gather2_call(x3, agg4, ids_pad)
    xs = xs_pad[:nc]
    cn0 = cn0_pad[:nc]

    a_n, ccnt = _tc_acfg_call(ce_r, nc)

    def wset(lyr):
        return (lyr["Wl"].T, lyr["Wr"].T, lyr["bl"].reshape(1, -1))

    cn3 = _tc_cfg_nbr_call(cn0, a_n, ccnt,
                           [wset(l) for l in params["cfg_nbr_gnn"]])
    y = _tc_config_head_call(
        node_config_feat, cn3, xs, params["prj_W"].T,
        params["prj_b"].reshape(1, -1), a_n, ccnt,
        [wset(l) for l in params["config_gnn"]],
        params["d1"].T, params["d2"].T, params["d3"].T)
    return y[:, 0, 0]
